# baseline (speedup 1.0000x reference)
import functools

import jax
import jax.numpy as jnp
from jax import lax
from jax.experimental import pallas as pl
from jax.experimental.pallas import tpu as pltpu
from jax.experimental.pallas import tpu_sc as plsc

_VOCAB = 1_000_000
_E = 64
_B = 16384
_L = 50
_LP = 64
_LANES = 16

_NC = 2
_NS = 16
_NW = _NC * _NS
_BPW = _B // _NW
_C = 16
_NCH = _BPW // _C
_CL = _C * _L
_UPAD = 16


def _main_body(cpad_hbm, ctxpad_hbm, v_hbm, u_hbm, out_hbm,
               cbuf_v, ctxbuf_v, cidx_v, ctxbm_v,
               vrows_v, urows_v, out_v, sem_v, sem_u):
    wid = lax.axis_index("s") * _NC + lax.axis_index("c")
    lane = lax.iota(jnp.int32, _LANES)
    zero16 = lane * 0

    def chunk_body(c, carry):
        base = wid * _BPW + c * _C
        pltpu.sync_copy(cpad_hbm.at[pl.ds(base, _C), :], cbuf_v)
        pltpu.sync_copy(ctxpad_hbm.at[pl.ds(base, _C), :], ctxbuf_v)
        cidx_v[pl.ds(0, 16)] = plsc.load_gather(cbuf_v, [lane, zero16])
        cp_v = pltpu.async_copy(v_hbm.at[cidx_v], vrows_v, sem_v)
        for b in range(_C):
            for g in range(4):
                vec = ctxbuf_v[b, pl.ds(g * 16, 16)]
                tgt = lane + (b * _L + g * 16)
                if g < 3:
                    plsc.store_scatter(ctxbm_v, [tgt], vec)
                else:
                    plsc.store_scatter(ctxbm_v, [tgt], vec, mask=lane < 2)
        cps = []
        for t in range(6):
            cps.append(pltpu.async_copy(
                u_hbm.at[ctxbm_v.at[pl.ds(t * 128, 128)]],
                urows_v.at[pl.ds(t * 128, 128)], sem_u))
        cps.append(pltpu.async_copy(
            u_hbm.at[ctxbm_v.at[pl.ds(768, 32)]],
            urows_v.at[pl.ds(768, 32)], sem_u))
        cp_v.wait()
        for cp in cps:
            cp.wait()

        for b in range(_C):
            v0 = vrows_v[b, pl.ds(0, 16)]
            v1 = vrows_v[b, pl.ds(16, 16)]
            v2 = vrows_v[b, pl.ds(32, 16)]
            v3 = vrows_v[b, pl.ds(48, 16)]
            zero = jnp.zeros((_LANES,), jnp.float32)

            def jbody(j, rs, b=b, v0=v0, v1=v1, v2=v2, v3=v3):
                out = []
                for g in range(4):
                    row = b * _L + g * 16 + j
                    acc = urows_v[row, pl.ds(0, 16)] * v0
                    acc = acc + urows_v[row, pl.ds(16, 16)] * v1
                    acc = acc + urows_v[row, pl.ds(32, 16)] * v2
                    acc = acc + urows_v[row, pl.ds(48, 16)] * v3
                    s = jnp.sum(acc)
                    out.append(jnp.where(lane == j, s, rs[g]))
                return tuple(out)

            r = lax.fori_loop(0, _LANES, jbody, (zero, zero, zero, zero))
            for g in range(4):
                out_v[pl.ds(b * _LP + g * 16, 16)] = r[g]

        pltpu.sync_copy(out_v, out_hbm.at[pl.ds(base * _LP, _C * _LP)])
        return carry

    lax.fori_loop(0, _NCH, chunk_body, 0)


def _sc_call(centerpad, ctxpad, v_w, u_w):
    mesh = plsc.VectorSubcoreMesh(core_axis_name="c", subcore_axis_name="s")
    main = functools.partial(
        pl.kernel,
        mesh=mesh,
        compiler_params=pltpu.CompilerParams(
            needs_layout_passes=False, use_tc_tiling_on_sc=False),
        out_type=jax.ShapeDtypeStruct((_B * _LP,), jnp.float32),
        scratch_types=[
            pltpu.VMEM((_C, 2 * _E), jnp.int32),
            pltpu.VMEM((_C, 2 * _E), jnp.int32),
            pltpu.VMEM((_C,), jnp.int32),
            pltpu.VMEM((_CL,), jnp.int32),
            pltpu.VMEM((_C, _E), jnp.float32),
            pltpu.VMEM((_CL + _UPAD, _E), jnp.float32),
            pltpu.VMEM((_C * _LP,), jnp.float32),
            pltpu.SemaphoreType.DMA,
            pltpu.SemaphoreType.DMA,
        ],
    )(_main_body)
    return main(centerpad, ctxpad, v_w, u_w)


def kernel(center, contexts_and_negatives, embed_v_weight, embed_u_weight):
    centerpad = jnp.pad(center, ((0, 0), (0, 2 * _E - 1)))
    ctxpad = jnp.pad(contexts_and_negatives, ((0, 0), (0, 2 * _E - _L)))
    out = _sc_call(centerpad, ctxpad, embed_v_weight, embed_u_weight)
    return out.reshape(_B, _LP)[:, :_L].reshape(_B, 1, _L)


# speedup vs baseline: 1.1830x; 1.1830x over previous
import functools

import jax
import jax.numpy as jnp
from jax import lax
from jax.experimental import pallas as pl
from jax.experimental.pallas import tpu as pltpu
from jax.experimental.pallas import tpu_sc as plsc

_VOCAB = 1_000_000
_E = 64
_B = 16384
_L = 50
_LP = 64
_LANES = 16

_NC = 2
_NS = 16
_NW = _NC * _NS
_BPW = _B // _NW
_C = 16
_NCH = _BPW // _C
_CL = _C * _L
_UPAD = 16


def _vgather_body(cpad_hbm, v_hbm, vg_hbm, cbuf_v, sem):
    wid = lax.axis_index("s") * _NC + lax.axis_index("c")

    def mini_body(m, carry):
        base = wid * _BPW + m * _C
        pltpu.sync_copy(cpad_hbm.at[pl.ds(base, _C), :], cbuf_v)
        descs = []
        for b in range(_C):
            idx = cbuf_v[b, pl.ds(0, 16)][0]
            descs.append(pltpu.async_copy(
                v_hbm.at[pl.ds(idx, 1), :],
                vg_hbm.at[pl.ds(base + b, 1), :], sem))
        for d in descs:
            d.wait()
        return carry

    lax.fori_loop(0, _NCH, mini_body, 0)


def _main_body(ctxpad_hbm, vg_hbm, u_hbm, out_hbm,
               ctxbuf_v, vbuf_v, ctxbm_v, urows_v, out_v, sem_u):
    wid = lax.axis_index("s") * _NC + lax.axis_index("c")
    lane = lax.iota(jnp.int32, _LANES)

    def chunk_body(c, carry):
        base = wid * _BPW + c * _C
        pltpu.sync_copy(ctxpad_hbm.at[pl.ds(base, _C), :], ctxbuf_v)
        pltpu.sync_copy(vg_hbm.at[pl.ds(base, _C), :], vbuf_v)
        for b in range(_C):
            for g in range(4):
                vec = ctxbuf_v[b, pl.ds(g * 16, 16)]
                tgt = lane + (b * _L + g * 16)
                if g < 3:
                    plsc.store_scatter(ctxbm_v, [tgt], vec)
                else:
                    plsc.store_scatter(ctxbm_v, [tgt], vec, mask=lane < 2)
        cps = []
        for t in range(6):
            cps.append(pltpu.async_copy(
                u_hbm.at[ctxbm_v.at[pl.ds(t * 128, 128)]],
                urows_v.at[pl.ds(t * 128, 128)], sem_u))
        cps.append(pltpu.async_copy(
            u_hbm.at[ctxbm_v.at[pl.ds(768, 32)]],
            urows_v.at[pl.ds(768, 32)], sem_u))
        for cp in cps:
            cp.wait()

        for b in range(_C):
            v0 = vbuf_v[b, pl.ds(0, 16)]
            v1 = vbuf_v[b, pl.ds(16, 16)]
            v2 = vbuf_v[b, pl.ds(32, 16)]
            v3 = vbuf_v[b, pl.ds(48, 16)]
            zero = jnp.zeros((_LANES,), jnp.float32)

            def jbody(j, rs, b=b, v0=v0, v1=v1, v2=v2, v3=v3):
                out = []
                for g in range(4):
                    row = b * _L + g * 16 + j
                    acc = urows_v[row, pl.ds(0, 16)] * v0
                    acc = acc + urows_v[row, pl.ds(16, 16)] * v1
                    acc = acc + urows_v[row, pl.ds(32, 16)] * v2
                    acc = acc + urows_v[row, pl.ds(48, 16)] * v3
                    s = jnp.sum(acc)
                    out.append(jnp.where(lane == j, s, rs[g]))
                return tuple(out)

            r = lax.fori_loop(0, _LANES, jbody, (zero, zero, zero, zero))
            for g in range(4):
                out_v[pl.ds(b * _LP + g * 16, 16)] = r[g]

        pltpu.sync_copy(out_v, out_hbm.at[pl.ds(base * _LP, _C * _LP)])
        return carry

    lax.fori_loop(0, _NCH, chunk_body, 0)


def _sc_call(centerpad, ctxpad, v_w, u_w):
    mesh = plsc.VectorSubcoreMesh(core_axis_name="c", subcore_axis_name="s")
    vgather = functools.partial(
        pl.kernel,
        mesh=mesh,
        compiler_params=pltpu.CompilerParams(
            needs_layout_passes=False, use_tc_tiling_on_sc=True),
        out_type=jax.ShapeDtypeStruct((_B, _E), jnp.float32),
        scratch_types=[
            pltpu.VMEM((_C, 2 * _E), jnp.int32),
            pltpu.SemaphoreType.DMA,
        ],
    )(_vgather_body)
    vg = vgather(centerpad, v_w)

    main = functools.partial(
        pl.kernel,
        mesh=mesh,
        compiler_params=pltpu.CompilerParams(
            needs_layout_passes=False, use_tc_tiling_on_sc=False),
        out_type=jax.ShapeDtypeStruct((_B * _LP,), jnp.float32),
        scratch_types=[
            pltpu.VMEM((_C, 2 * _E), jnp.int32),
            pltpu.VMEM((_C, _E), jnp.float32),
            pltpu.VMEM((_CL,), jnp.int32),
            pltpu.VMEM((_CL + _UPAD, _E), jnp.float32),
            pltpu.VMEM((_C * _LP,), jnp.float32),
            pltpu.SemaphoreType.DMA,
        ],
    )(_main_body)
    return main(ctxpad, vg, u_w)


def kernel(center, contexts_and_negatives, embed_v_weight, embed_u_weight):
    centerpad = jnp.pad(center, ((0, 0), (0, 2 * _E - 1)))
    ctxpad = jnp.pad(contexts_and_negatives, ((0, 0), (0, 2 * _E - _L)))
    out = _sc_call(centerpad, ctxpad, embed_v_weight, embed_u_weight)
    return out.reshape(_B, _LP)[:, :_L].reshape(_B, 1, _L)
